# Initial kernel scaffold; baseline (speedup 1.0000x reference)
import jax
import jax.numpy as jnp
from jax.experimental import pallas as pl


def kernel(x, out_degree, num_total_graphs, out_degree_table, graph_token):
    raise NotImplementedError("write your pallas kernel here")


# speedup vs baseline: 1.3980x; 1.3980x over previous
import functools

import jax
import jax.numpy as jnp
from jax import lax
from jax.experimental import pallas as pl
from jax.experimental.pallas import tpu as pltpu
from jax.experimental.pallas import tpu_sc as plsc

N = 50000
D = 512
V = 512
G = 64

C = 80
NB = N // C
NW = 32
MAX_T = (NB + NW - 1) // NW
LANES = 16


def _body(x_hbm, idx_hbm, table_hbm, tok_hbm, out_hbm,
          idx_v, x_buf, g_buf, tok_idx, tok_buf, sem, tok_sem):
    wid = lax.axis_index("s") * 2 + lax.axis_index("c")

    @pl.when(wid == NW - 1)
    def _tok():
        for j in range(G // LANES):
            tok_idx[pl.ds(j * LANES, LANES)] = jnp.zeros((LANES,), jnp.int32)
        pltpu.async_copy(tok_hbm.at[tok_idx], tok_buf, tok_sem).wait()
        pltpu.sync_copy(tok_buf, out_hbm.at[pl.ds(0, G), :])

    def step(t, carry):
        b = wid + NW * t

        @pl.when(b < NB)
        def _blk():
            base = b * C
            pltpu.sync_copy(idx_hbm.at[pl.ds(base, C)], idx_v)
            gcp = pltpu.async_copy(table_hbm.at[idx_v], g_buf, sem)
            pltpu.sync_copy(x_hbm.at[pl.ds(base, C), :], x_buf)
            gcp.wait()

            def row(r, c2):
                for j in range(D // LANES):
                    sl = pl.ds(j * LANES, LANES)
                    x_buf[r, sl] = x_buf[r, sl] + g_buf[r, sl]
                return c2

            lax.fori_loop(0, C, row, 0)
            pltpu.sync_copy(x_buf, out_hbm.at[pl.ds(G + base, C), :])

        return carry

    lax.fori_loop(0, MAX_T, step, 0)


@jax.jit
def _run(x, out_degree, out_degree_table, graph_token):
    mesh = plsc.VectorSubcoreMesh(core_axis_name="c", subcore_axis_name="s")
    fn = pl.kernel(
        _body,
        out_type=jax.ShapeDtypeStruct((N + G, D), jnp.float32),
        mesh=mesh,
        scratch_types=[
            pltpu.VMEM((C,), jnp.int32),
            pltpu.VMEM((C, D), jnp.float32),
            pltpu.VMEM((C, D), jnp.float32),
            pltpu.VMEM((G,), jnp.int32),
            pltpu.VMEM((G, D), jnp.float32),
            pltpu.SemaphoreType.DMA,
            pltpu.SemaphoreType.DMA,
        ],
    )
    return fn(x, out_degree, out_degree_table, graph_token)


def kernel(x, out_degree, num_total_graphs, out_degree_table, graph_token):
    del num_total_graphs
    return _run(x, out_degree, out_degree_table, graph_token)


# speedup vs baseline: 1.8801x; 1.3448x over previous
import jax
import jax.numpy as jnp
from jax import lax
from jax.experimental import pallas as pl
from jax.experimental.pallas import tpu as pltpu
from jax.experimental.pallas import tpu_sc as plsc

N = 50000
D = 512
V = 512
G = 64

C = 40
NW = 32
T_BIG = 40
T_SMALL = 39
LANES = 16


def _body(x_hbm, idx_hbm, table_hbm, tok_hbm, out_hbm,
          idx_all, x0, x1, g0, g1, tok_idx, tok_buf,
          sg0, sg1, sx0, sx1, so0, so1, tok_sem):
    wid = lax.axis_index("s") * 2 + lax.axis_index("c")
    big = wid < 2
    start = jnp.where(big, wid * (C * T_BIG),
                      2 * (C * T_BIG) + (wid - 2) * (C * T_SMALL))

    @pl.when(wid == NW - 1)
    def _tok():
        for j in range(G // LANES):
            tok_idx[pl.ds(j * LANES, LANES)] = jnp.zeros((LANES,), jnp.int32)
        pltpu.async_copy(tok_hbm.at[tok_idx], tok_buf, tok_sem).wait()
        pltpu.sync_copy(tok_buf, out_hbm.at[pl.ds(0, G), :])

    pltpu.sync_copy(idx_hbm.at[pl.ds(start, C * T_SMALL)],
                    idx_all.at[pl.ds(0, C * T_SMALL)])

    @pl.when(big)
    def _tail_idx():
        pltpu.sync_copy(idx_hbm.at[pl.ds(start + C * T_SMALL, C)],
                        idx_all.at[pl.ds(C * T_SMALL, C)])

    xb = (x0, x1)
    gb = (g0, g1)
    sg = (sg0, sg1)
    sx = (sx0, sx1)
    so = (so0, so1)

    def start_loads(t):
        k = t % 2
        pltpu.async_copy(table_hbm.at[idx_all.at[pl.ds(t * C, C)]], gb[k], sg[k])
        pltpu.async_copy(x_hbm.at[pl.ds(start + t * C, C), :], xb[k], sx[k])

    def wait_loads(t):
        k = t % 2
        pltpu.make_async_copy(table_hbm.at[idx_all.at[pl.ds(t * C, C)]],
                              gb[k], sg[k]).wait()
        pltpu.make_async_copy(x_hbm.at[pl.ds(start + t * C, C), :],
                              xb[k], sx[k]).wait()

    def out_copy(t):
        k = t % 2
        return pltpu.make_async_copy(
            xb[k], out_hbm.at[pl.ds(G + start + t * C, C), :], so[k])

    start_loads(0)

    for t in range(T_BIG):
        k = t % 2
        guard = pl.when(big) if (t >= T_SMALL) else (lambda f: f())

        def _iter(t=t, k=k):
            if t >= 2:
                out_copy(t - 2).wait()
            if t + 1 < T_BIG:
                if t + 1 >= T_SMALL:
                    @pl.when(big)
                    def _ld():
                        start_loads(t + 1)
                else:
                    start_loads(t + 1)
            wait_loads(t)

            def row(r, c2):
                for j in range(D // LANES):
                    sl = pl.ds(j * LANES, LANES)
                    xb[k][r, sl] = xb[k][r, sl] + gb[k][r, sl]
                return c2

            lax.fori_loop(0, C, row, 0)
            out_copy(t).start()

        guard(_iter)

    @pl.when(big)
    def _drain_big():
        out_copy(T_BIG - 2).wait()
        out_copy(T_BIG - 1).wait()

    @pl.when(jnp.logical_not(big))
    def _drain_small():
        out_copy(T_SMALL - 2).wait()
        out_copy(T_SMALL - 1).wait()


@jax.jit
def _run(x, out_degree, out_degree_table, graph_token):
    mesh = plsc.VectorSubcoreMesh(core_axis_name="c", subcore_axis_name="s")
    fn = pl.kernel(
        _body,
        out_type=jax.ShapeDtypeStruct((N + G, D), jnp.float32),
        mesh=mesh,
        scratch_types=[
            pltpu.VMEM((C * T_BIG,), jnp.int32),
            pltpu.VMEM((C, D), jnp.float32),
            pltpu.VMEM((C, D), jnp.float32),
            pltpu.VMEM((C, D), jnp.float32),
            pltpu.VMEM((C, D), jnp.float32),
            pltpu.VMEM((G,), jnp.int32),
            pltpu.VMEM((G, D), jnp.float32),
            pltpu.SemaphoreType.DMA,
            pltpu.SemaphoreType.DMA,
            pltpu.SemaphoreType.DMA,
            pltpu.SemaphoreType.DMA,
            pltpu.SemaphoreType.DMA,
            pltpu.SemaphoreType.DMA,
            pltpu.SemaphoreType.DMA,
        ],
    )
    return fn(x, out_degree, out_degree_table, graph_token)


def kernel(x, out_degree, num_total_graphs, out_degree_table, graph_token):
    del num_total_graphs
    return _run(x, out_degree, out_degree_table, graph_token)


# speedup vs baseline: 1.9971x; 1.0623x over previous
import jax
import jax.numpy as jnp
from jax import lax
from jax.experimental import pallas as pl
from jax.experimental.pallas import tpu as pltpu
from jax.experimental.pallas import tpu_sc as plsc

N = 50000
D = 512
V = 512
G = 64

C = 40
NW = 32
T_BIG = 40
T_SMALL = 39
LANES = 16


def _body(x_hbm, idx_hbm, table_hbm, tok_hbm, out_hbm,
          idx_all, x0, x1, g0, g1, tok_idx, tok_buf,
          sg0, sg1, sx0, sx1, so0, so1, tok_sem):
    wid = lax.axis_index("s") * 2 + lax.axis_index("c")
    big = wid < 2
    nblocks = jnp.where(big, T_BIG, T_SMALL)
    start = jnp.where(big, wid * (C * T_BIG),
                      2 * (C * T_BIG) + (wid - 2) * (C * T_SMALL))

    @pl.when(wid == NW - 1)
    def _tok():
        for j in range(G // LANES):
            tok_idx[pl.ds(j * LANES, LANES)] = jnp.zeros((LANES,), jnp.int32)
        pltpu.async_copy(tok_hbm.at[tok_idx], tok_buf, tok_sem).wait()
        pltpu.sync_copy(tok_buf, out_hbm.at[pl.ds(0, G), :])

    pltpu.sync_copy(idx_hbm.at[pl.ds(start, C * T_SMALL)],
                    idx_all.at[pl.ds(0, C * T_SMALL)])

    @pl.when(big)
    def _tail_idx():
        pltpu.sync_copy(idx_hbm.at[pl.ds(start + C * T_SMALL, C)],
                        idx_all.at[pl.ds(C * T_SMALL, C)])

    xb = (x0, x1)
    gb = (g0, g1)
    sg = (sg0, sg1)
    sx = (sx0, sx1)
    so = (so0, so1)

    def start_loads(t, k):
        pltpu.async_copy(table_hbm.at[idx_all.at[pl.ds(t * C, C)]], gb[k], sg[k])
        pltpu.async_copy(x_hbm.at[pl.ds(start + t * C, C), :], xb[k], sx[k])

    def wait_loads(t, k):
        pltpu.make_async_copy(table_hbm.at[idx_all.at[pl.ds(t * C, C)]],
                              gb[k], sg[k]).wait()
        pltpu.make_async_copy(x_hbm.at[pl.ds(start + t * C, C), :],
                              xb[k], sx[k]).wait()

    def out_copy(t, k):
        return pltpu.make_async_copy(
            xb[k], out_hbm.at[pl.ds(G + start + t * C, C), :], so[k])

    start_loads(0, 0)

    def pair(tp, carry):
        for par in range(2):
            t = 2 * tp + par
            k = par

            @pl.when(t < nblocks)
            def _it(t=t, k=k):
                @pl.when(t >= 2)
                def _w():
                    out_copy(t - 2, k).wait()

                @pl.when(t + 1 < nblocks)
                def _ld():
                    start_loads(t + 1, 1 - k)

                wait_loads(t, k)

                @plsc.parallel_loop(0, C, step=1, unroll=4)
                def _row(r):
                    for j in range(D // LANES):
                        sl = pl.ds(j * LANES, LANES)
                        xb[k][r, sl] = xb[k][r, sl] + gb[k][r, sl]

                out_copy(t, k).start()

        return carry

    lax.fori_loop(0, (T_BIG + 1) // 2, pair, 0)

    for k in range(2):
        t_k = jnp.where((nblocks - 1) % 2 == k, nblocks - 1, nblocks - 2)
        out_copy(t_k, k).wait()


@jax.jit
def _run(x, out_degree, out_degree_table, graph_token):
    mesh = plsc.VectorSubcoreMesh(core_axis_name="c", subcore_axis_name="s")
    fn = pl.kernel(
        _body,
        out_type=jax.ShapeDtypeStruct((N + G, D), jnp.float32),
        mesh=mesh,
        scratch_types=[
            pltpu.VMEM((C * T_BIG,), jnp.int32),
            pltpu.VMEM((C, D), jnp.float32),
            pltpu.VMEM((C, D), jnp.float32),
            pltpu.VMEM((C, D), jnp.float32),
            pltpu.VMEM((C, D), jnp.float32),
            pltpu.VMEM((G,), jnp.int32),
            pltpu.VMEM((G, D), jnp.float32),
            pltpu.SemaphoreType.DMA,
            pltpu.SemaphoreType.DMA,
            pltpu.SemaphoreType.DMA,
            pltpu.SemaphoreType.DMA,
            pltpu.SemaphoreType.DMA,
            pltpu.SemaphoreType.DMA,
            pltpu.SemaphoreType.DMA,
        ],
    )
    return fn(x, out_degree, out_degree_table, graph_token)


def kernel(x, out_degree, num_total_graphs, out_degree_table, graph_token):
    del num_total_graphs
    return _run(x, out_degree, out_degree_table, graph_token)
